# baseline (speedup 1.0000x reference)
import jax
import jax.numpy as jnp
from jax import lax
from jax.experimental import pallas as pl

NG = 4762
NC = 847
D = 256
E_DEC = 200000


def _tc_pre_body(gene_ref, cell_ref, wg_ref, wc_ref, dog_ref, doc_ref,
                 xg_ref, xc_ref):
    sg = lax.rsqrt(jnp.maximum(dog_ref[...], 1.0))
    xg_ref[...] = jnp.dot(gene_ref[...] * sg[:, None], wg_ref[...],
                          preferred_element_type=jnp.float32)
    sc = lax.rsqrt(jnp.maximum(doc_ref[...], 1.0))
    xc_ref[...] = jnp.dot(cell_ref[...] * sc[:, None], wc_ref[...],
                          preferred_element_type=jnp.float32)


def _tc_post_body(aggc_ref, aggg_ref, dic_ref, dig_ref, bgc_ref, bcg_ref,
                  wp_ref, bp_ref, hc_ref, hg_ref, sc_ref, sg_ref):
    ric = lax.rsqrt(jnp.maximum(dic_ref[...], 1.0))
    h_c = jnp.maximum(aggc_ref[...] * ric[:, None] + bgc_ref[...], 0.0)
    hc_ref[...] = h_c
    sc_ref[...] = jnp.dot(h_c, wp_ref[D:, :],
                          preferred_element_type=jnp.float32) + bp_ref[...]
    rig = lax.rsqrt(jnp.maximum(dig_ref[...], 1.0))
    h_g = jnp.maximum(aggg_ref[...] * rig[:, None] + bcg_ref[...], 0.0)
    hg_ref[...] = h_g
    sg_ref[...] = jnp.dot(h_g, wp_ref[:D, :],
                          preferred_element_type=jnp.float32)


def kernel(enc_g2c_src, enc_g2c_dst, enc_c2g_src, enc_c2g_dst, dec_src,
           dec_dst, gene_emb, cell_emb, W_g2c, b_g2c, W_c2g, b_c2g, Wp, bp):
    dog = jnp.bincount(enc_g2c_src, length=NG).astype(jnp.float32)
    dic = jnp.bincount(enc_g2c_dst, length=NC).astype(jnp.float32)
    doc = jnp.bincount(enc_c2g_src, length=NC).astype(jnp.float32)
    dig = jnp.bincount(enc_c2g_dst, length=NG).astype(jnp.float32)

    xg, xc = pl.pallas_call(
        _tc_pre_body,
        out_shape=(jax.ShapeDtypeStruct((NG, D), jnp.float32),
                   jax.ShapeDtypeStruct((NC, D), jnp.float32)),
    )(gene_emb, cell_emb, W_g2c, W_c2g, dog, doc)

    aggc = jnp.zeros((NC, D), jnp.float32).at[enc_g2c_dst].add(
        jnp.take(xg, enc_g2c_src, axis=0))
    aggg = jnp.zeros((NG, D), jnp.float32).at[enc_c2g_dst].add(
        jnp.take(xc, enc_c2g_src, axis=0))

    h_cell, h_gene, sc_t, sg_t = pl.pallas_call(
        _tc_post_body,
        out_shape=(jax.ShapeDtypeStruct((NC, D), jnp.float32),
                   jax.ShapeDtypeStruct((NG, D), jnp.float32),
                   jax.ShapeDtypeStruct((NC, 1), jnp.float32),
                   jax.ShapeDtypeStruct((NG, 1), jnp.float32)),
    )(aggc, aggg, dic, dig, b_g2c.reshape(1, D), b_c2g.reshape(1, D),
      Wp, bp.reshape(1, 1))

    score = (sg_t[:, 0][dec_src] + sc_t[:, 0][dec_dst])[:, None]
    return (score, h_gene, h_cell)
